# baseline (speedup 1.0000x reference)
import jax
import jax.numpy as jnp
from jax.experimental import pallas as pl
from jax.experimental.pallas import tpu as pltpu

B, N, DIN, DOUT = 4, 2048, 128, 128
BM = 512


def _gcn_body(x_ref, w_ref, adj_ref, bias_ref, out_ref):
    j = pl.program_id(1)

    sup = jnp.dot(
        x_ref[0], w_ref[...], preferred_element_type=jnp.float32
    ).astype(jnp.bfloat16)
    partial = jax.lax.dot_general(
        adj_ref[0].astype(jnp.bfloat16),
        sup,
        (((0,), (0,)), ((), ())),
        preferred_element_type=jnp.float32,
    )

    @pl.when(j == 0)
    def _():
        out_ref[0] = partial + bias_ref[...]

    @pl.when(j != 0)
    def _():
        out_ref[0] += partial


@jax.jit
def kernel(input, adj, weight, bias):
    bias2d = bias.reshape(1, DOUT)
    grid = (B, N // BM)
    return pl.pallas_call(
        _gcn_body,
        grid=grid,
        in_specs=[
            pl.BlockSpec((1, BM, DIN), lambda b, j: (b, j, 0)),
            pl.BlockSpec((DIN, DOUT), lambda b, j: (0, 0)),
            pl.BlockSpec((1, BM, N), lambda b, j: (b, j, 0)),
            pl.BlockSpec((1, DOUT), lambda b, j: (0, 0)),
        ],
        out_specs=pl.BlockSpec((1, N, DOUT), lambda b, j: (b, 0, 0)),
        out_shape=jax.ShapeDtypeStruct((B, N, DOUT), jnp.float32),
        compiler_params=pltpu.CompilerParams(
            dimension_semantics=("arbitrary", "arbitrary"),
        ),
    )(input, weight, adj, bias2d)
